# baseline (speedup 1.0000x reference)
import functools

import jax
import jax.numpy as jnp
from jax.experimental import pallas as pl
from jax.experimental.pallas import tpu as pltpu
from jax.experimental.pallas import tpu_sc as plsc

B = 8
C = 32
T = 48 * 48
K = 9
KPAD = 16

R = 256
NT = T // R

NC = 2
NS = 16
NW = NC * NS
NTOT = B * T * K
PER_W = NTOT // NW
CH = 96
NCH = PER_W // CH


def _topk_body(xf_ref, xr_ref, idx_ref, xt_ref):
    b = pl.program_id(0)
    xfb = xf_ref[0]
    norm = jnp.sqrt(jnp.sum(xfb * xfb, axis=0, keepdims=True))
    xn = xfb / jnp.maximum(norm, 1e-12)
    rows = xr_ref[0]
    norm_r = jnp.sqrt(jnp.sum(rows * rows, axis=0, keepdims=True))
    rows_n = rows / jnp.maximum(norm_r, 1e-12)
    xt_ref[0] = rows.T

    sim = jax.lax.dot_general(
        xn, rows_n, (((0,), (0,)), ((), ())),
        preferred_element_type=jnp.float32)
    sim = jnp.clip(sim, -1.0, 1.0)

    iota_f = jax.lax.broadcasted_iota(jnp.int32, (2, T), 1).astype(jnp.float32)
    row_id = jax.lax.broadcasted_iota(jnp.int32, (2, T), 0)
    lhs2 = jnp.where(row_id == 0, iota_f, 1.0)

    m = jnp.max(sim, axis=0, keepdims=True)
    picks = []
    for k in range(K):
        ge = sim >= m
        gef = jnp.where(ge, 1.0, 0.0)
        ms = jax.lax.dot_general(
            lhs2, gef, (((1,), (0,)), ((), ())),
            precision=jax.lax.Precision.HIGHEST,
            preferred_element_type=jnp.float32)
        ik = (ms[0:1] / ms[1:2] + 0.5).astype(jnp.int32)
        picks.append(ik)
        if k < K - 1:
            sim = jnp.where(ge, -3.0, sim)
            m = jnp.max(sim, axis=0, keepdims=True)
    idx = jnp.concatenate(picks + [jnp.zeros((KPAD - K, R), jnp.int32)], axis=0)
    idx_ref[0] = idx + b * T


_topk_call = pl.pallas_call(
    _topk_body,
    grid=(B, NT),
    in_specs=[pl.BlockSpec((1, C, T), lambda b, rt: (b, 0, 0)),
              pl.BlockSpec((1, C, R), lambda b, rt: (b, 0, rt))],
    out_specs=[
        pl.BlockSpec((1, KPAD, R), lambda b, rt: (b, 0, rt)),
        pl.BlockSpec((1, R, C), lambda b, rt: (b, rt, 0)),
    ],
    out_shape=[
        jax.ShapeDtypeStruct((B, KPAD, T), jnp.int32),
        jax.ShapeDtypeStruct((B, T, C), jnp.float32),
    ],
)


def _gather_body(tab_ref, idx_ref, out_ref, idx_v, rows_v, sem):
    c = jax.lax.axis_index("c")
    s = jax.lax.axis_index("s")
    wid = s * NC + c
    pltpu.sync_copy(idx_ref.at[wid], idx_v)

    def chunk(j, carry):
        pltpu.async_copy(tab_ref.at[idx_v.at[j]], rows_v, sem).wait()
        pltpu.sync_copy(rows_v, out_ref.at[pl.ds(wid * PER_W + j * CH, CH)])
        return carry

    jax.lax.fori_loop(0, NCH, chunk, 0)


@functools.cache
def _make_gather_call():
    return pl.kernel(
        _gather_body,
        out_type=jax.ShapeDtypeStruct((NTOT, C), jnp.float32),
        mesh=plsc.VectorSubcoreMesh(core_axis_name="c", subcore_axis_name="s",
                                    num_cores=NC, num_subcores=NS),
        scratch_types=[
            pltpu.VMEM((NCH, CH), jnp.int32),
            pltpu.VMEM((CH, C), jnp.float32),
            pltpu.SemaphoreType.DMA,
        ],
        compiler_params=pltpu.CompilerParams(use_tc_tiling_on_sc=False),
    )


def _conv_body(p_ref, w_ref, b_ref, o_ref):
    k = pl.program_id(1)
    contrib = jax.lax.dot_general(
        w_ref[0], p_ref[0], (((1,), (1,)), ((), ())),
        preferred_element_type=jnp.float32)

    @pl.when(k == 0)
    def _():
        o_ref[0] = contrib + b_ref[...]

    @pl.when(k != 0)
    def _():
        o_ref[0] = o_ref[0] + contrib


_conv_call = pl.pallas_call(
    _conv_body,
    grid=(B, K),
    in_specs=[
        pl.BlockSpec((1, T, C), lambda b, k: (b * K + k, 0, 0)),
        pl.BlockSpec((1, C, C), lambda b, k: (k, 0, 0)),
        pl.BlockSpec((C, 1), lambda b, k: (0, 0)),
    ],
    out_specs=pl.BlockSpec((1, C, T), lambda b, k: (b, 0, 0)),
    out_shape=jax.ShapeDtypeStruct((B, C, T), jnp.float32),
)


def kernel(x, W, b):
    xf = x.reshape(B, C, T)
    idx_g, xt = _topk_call(xf, xf)
    idx3 = idx_g[:, :K, :].reshape(NW, NCH, CH)
    prime = _make_gather_call()(xt.reshape(B * T, C), idx3)
    p3 = prime.reshape(B * K, T, C)
    w9 = W.transpose(2, 0, 1)
    out = _conv_call(p3, w9, b.reshape(C, 1))
    return out.reshape(B, C, 48, 48)


# speedup vs baseline: 2.2703x; 2.2703x over previous
import functools

import jax
import jax.numpy as jnp
from jax.experimental import pallas as pl
from jax.experimental.pallas import tpu as pltpu
from jax.experimental.pallas import tpu_sc as plsc

B = 8
C = 32
T = 48 * 48
K = 9
KPAD = 16

R = 256
NT = T // R

NC = 2
NS = 16
NW = NC * NS
NTOT = B * T * K
PER_W = NTOT // NW
CH = 96
NCH = PER_W // CH


def _topk_body(xf_ref, xr_ref, idx_ref, xt_ref):
    b = pl.program_id(0)
    xfb = xf_ref[0]
    norm = jnp.sqrt(jnp.sum(xfb * xfb, axis=0, keepdims=True))
    xn = xfb / jnp.maximum(norm, 1e-12)
    rows = xr_ref[0]
    norm_r = jnp.sqrt(jnp.sum(rows * rows, axis=0, keepdims=True))
    rows_n = rows / jnp.maximum(norm_r, 1e-12)
    xt_ref[0] = rows.T

    sim = jax.lax.dot_general(
        xn, rows_n, (((0,), (0,)), ((), ())),
        preferred_element_type=jnp.float32)
    sim = jnp.clip(sim, -1.0, 1.0)

    iota_i = jax.lax.broadcasted_iota(jnp.int32, (3, T), 1)
    row_id = jax.lax.broadcasted_iota(jnp.int32, (3, T), 0)
    lhs3 = jnp.where(
        row_id == 0, iota_i // 256,
        jnp.where(row_id == 1, iota_i % 256, 1)).astype(jnp.bfloat16)

    m = jnp.max(sim, axis=0, keepdims=True)
    picks = []
    for k in range(K):
        ge = sim >= m
        gef = jnp.where(ge, 1.0, 0.0).astype(jnp.bfloat16)
        ms = jax.lax.dot_general(
            lhs3, gef, (((1,), (0,)), ((), ())),
            preferred_element_type=jnp.float32)
        ikf = (ms[0:1] * 256.0 + ms[1:2]) / ms[2:3]
        ik = jnp.clip((ikf + 0.5).astype(jnp.int32), 0, T - 1)
        picks.append(ik)
        if k < K - 1:
            sim = jnp.where(ge, -3.0, sim)
            m = jnp.max(sim, axis=0, keepdims=True)
    idx = jnp.concatenate(picks + [jnp.zeros((KPAD - K, R), jnp.int32)], axis=0)
    idx_ref[0] = idx + b * T


_topk_call = pl.pallas_call(
    _topk_body,
    grid=(B, NT),
    in_specs=[pl.BlockSpec((1, C, T), lambda b, rt: (b, 0, 0)),
              pl.BlockSpec((1, C, R), lambda b, rt: (b, 0, rt))],
    out_specs=[
        pl.BlockSpec((1, KPAD, R), lambda b, rt: (b, 0, rt)),
        pl.BlockSpec((1, R, C), lambda b, rt: (b, rt, 0)),
    ],
    out_shape=[
        jax.ShapeDtypeStruct((B, KPAD, T), jnp.int32),
        jax.ShapeDtypeStruct((B, T, C), jnp.float32),
    ],
)


def _gather_body(tab_ref, idx_ref, out_ref, idx_v, rows_v, sem):
    c = jax.lax.axis_index("c")
    s = jax.lax.axis_index("s")
    wid = s * NC + c
    pltpu.sync_copy(idx_ref.at[wid], idx_v)

    def chunk(j, carry):
        pltpu.async_copy(tab_ref.at[idx_v.at[j]], rows_v, sem).wait()
        pltpu.sync_copy(rows_v, out_ref.at[pl.ds(wid * PER_W + j * CH, CH)])
        return carry

    jax.lax.fori_loop(0, NCH, chunk, 0)


@functools.cache
def _make_gather_call():
    return pl.kernel(
        _gather_body,
        out_type=jax.ShapeDtypeStruct((NTOT, C), jnp.float32),
        mesh=plsc.VectorSubcoreMesh(core_axis_name="c", subcore_axis_name="s",
                                    num_cores=NC, num_subcores=NS),
        scratch_types=[
            pltpu.VMEM((NCH, CH), jnp.int32),
            pltpu.VMEM((CH, C), jnp.float32),
            pltpu.SemaphoreType.DMA,
        ],
        compiler_params=pltpu.CompilerParams(use_tc_tiling_on_sc=False),
    )


def _conv_body(p_ref, w_ref, b_ref, o_ref):
    k = pl.program_id(1)
    contrib = jax.lax.dot_general(
        w_ref[0], p_ref[0], (((1,), (1,)), ((), ())),
        preferred_element_type=jnp.float32)

    @pl.when(k == 0)
    def _():
        o_ref[0] = contrib + b_ref[...]

    @pl.when(k != 0)
    def _():
        o_ref[0] = o_ref[0] + contrib


_conv_call = pl.pallas_call(
    _conv_body,
    grid=(B, K),
    in_specs=[
        pl.BlockSpec((1, T, C), lambda b, k: (b * K + k, 0, 0)),
        pl.BlockSpec((1, C, C), lambda b, k: (k, 0, 0)),
        pl.BlockSpec((C, 1), lambda b, k: (0, 0)),
    ],
    out_specs=pl.BlockSpec((1, C, T), lambda b, k: (b, 0, 0)),
    out_shape=jax.ShapeDtypeStruct((B, C, T), jnp.float32),
)


def kernel(x, W, b):
    xf = x.reshape(B, C, T)
    idx_g, xt = _topk_call(xf, xf)
    idx3 = idx_g[:, :K, :].reshape(NW, NCH, CH)
    prime = _make_gather_call()(xt.reshape(B * T, C), idx3)
    p3 = prime.reshape(B * K, T, C)
    w9 = W.transpose(2, 0, 1)
    out = _conv_call(p3, w9, b.reshape(C, 1))
    return out.reshape(B, C, 48, 48)
